# baseline (speedup 1.0000x reference)
import jax
import jax.numpy as jnp
from jax import lax
from jax.experimental import pallas as pl
from jax.experimental.pallas import tpu as pltpu

_TILE = 1024
_NBUF = 6


def _router_kernel(x_hbm_ref, w1_ref, b1_ref, wv1_ref, bv1_ref,
                   w2t_ref, b2_ref, w3t_ref, b3_ref, wv2t_ref, bv2_ref,
                   logits_ref, values_ref,
                   xbuf_ref, wcat_ref, bcat_ref, sems):
    i = pl.program_id(0)
    n_steps = pl.num_programs(0)

    def _start_copy(blk):
        slot = lax.rem(blk, _NBUF)
        pltpu.make_async_copy(
            x_hbm_ref.at[pl.ds(blk * _TILE, _TILE), :],
            xbuf_ref.at[slot],
            sems.at[slot],
        ).start()

    @pl.when(i == 0)
    def _init():
        wcat_ref[:, :384] = w1_ref[...].astype(jnp.bfloat16)
        wcat_ref[:, 384:] = wv1_ref[...].astype(jnp.bfloat16)
        bcat_ref[0, :384] = b1_ref[...]
        bcat_ref[0, 384:] = bv1_ref[...]
        for blk in range(_NBUF - 1):
            _start_copy(jnp.int32(blk))

    nxt = i + _NBUF - 1

    @pl.when(nxt < n_steps)
    def _prefetch():
        _start_copy(nxt)

    slot = lax.rem(i, _NBUF)
    pltpu.make_async_copy(
        x_hbm_ref.at[pl.ds(i * _TILE, _TILE), :],
        xbuf_ref.at[slot],
        sems.at[slot],
    ).wait()

    x = xbuf_ref[slot].astype(jnp.bfloat16)
    h_all = jnp.maximum(
        jnp.dot(x, wcat_ref[...], preferred_element_type=jnp.float32)
        + bcat_ref[...],
        0.0,
    ).astype(jnp.bfloat16)
    h = h_all[:, :384]
    v = h_all[:, 384:]
    h2 = jnp.maximum(
        lax.dot_general(h, w2t_ref[...].astype(jnp.bfloat16),
                        (((1,), (1,)), ((), ())),
                        preferred_element_type=jnp.float32)
        + b2_ref[...],
        0.0,
    ).astype(jnp.bfloat16)
    logits_ref[0] = (
        lax.dot_general(w3t_ref[...].astype(jnp.bfloat16), h2,
                        (((1,), (1,)), ((), ())),
                        preferred_element_type=jnp.float32)
        + b3_ref[...]
    )
    values_ref[0] = (
        lax.dot_general(wv2t_ref[...].astype(jnp.bfloat16), v,
                        (((1,), (1,)), ((), ())),
                        preferred_element_type=jnp.float32)
        + bv2_ref[...]
    )


def kernel(hidden_states, W1, b1, W2, b2, W3, b3, Wv1, bv1, Wv2, bv2):
    B, S, H = hidden_states.shape
    N = B * S
    E = W3.shape[1]
    n_tiles = S // _TILE
    flat = hidden_states.reshape(N, H)
    logits_t, values_t = pl.pallas_call(
        _router_kernel,
        grid=(N // _TILE,),
        compiler_params=pltpu.CompilerParams(
            dimension_semantics=("arbitrary",),
            vmem_limit_bytes=100 * 1024 * 1024,
        ),
        in_specs=[
            pl.BlockSpec(memory_space=pl.ANY),
            pl.BlockSpec((H, H // 2), lambda i: (0, 0)),
            pl.BlockSpec((H // 2,), lambda i: (0,)),
            pl.BlockSpec((H, H // 2), lambda i: (0, 0)),
            pl.BlockSpec((H // 2,), lambda i: (0,)),
            pl.BlockSpec((H // 4, H // 2), lambda i: (0, 0)),
            pl.BlockSpec((1, H // 4), lambda i: (0, 0)),
            pl.BlockSpec((E, H // 4), lambda i: (0, 0)),
            pl.BlockSpec((E, 1), lambda i: (0, 0)),
            pl.BlockSpec((1, H // 2), lambda i: (0, 0)),
            pl.BlockSpec((1, 1), lambda i: (0, 0)),
        ],
        out_specs=[
            pl.BlockSpec((1, E, _TILE),
                         lambda i: (i // n_tiles, 0, i % n_tiles)),
            pl.BlockSpec((1, 1, _TILE),
                         lambda i: (i // n_tiles, 0, i % n_tiles)),
        ],
        out_shape=[
            jax.ShapeDtypeStruct((B, E, S), jnp.float32),
            jax.ShapeDtypeStruct((B, 1, S), jnp.float32),
        ],
        scratch_shapes=[
            pltpu.VMEM((_NBUF, _TILE, H), jnp.float32),
            pltpu.VMEM((H, H), jnp.bfloat16),
            pltpu.VMEM((1, H), jnp.float32),
            pltpu.SemaphoreType.DMA((_NBUF,)),
        ],
    )(flat, W1, b1, Wv1, bv1, W2.T, b2.reshape(1, -1),
      W3.T, b3.reshape(-1, 1), Wv2.T, bv2.reshape(-1, 1))
    logits = jnp.transpose(logits_t, (0, 2, 1))
    values = jnp.transpose(values_t, (0, 2, 1))
    return (logits, values)


# speedup vs baseline: 1.0684x; 1.0684x over previous
import jax
import jax.numpy as jnp
from jax import lax
from jax.experimental import pallas as pl
from jax.experimental.pallas import tpu as pltpu

_TILE = 2048


def _router_kernel(x_ref, w1_ref, b1_ref, wv1_ref, bv1_ref,
                   w2t_ref, b2_ref, w3t_ref, b3_ref, wv2t_ref, bv2_ref,
                   logits_ref, values_ref, wcat_ref, bcat_ref):
    @pl.when(pl.program_id(1) == 0)
    def _init():
        wcat_ref[:, :384] = w1_ref[...].astype(jnp.bfloat16)
        wcat_ref[:, 384:] = wv1_ref[...].astype(jnp.bfloat16)
        bcat_ref[0, :384] = b1_ref[...]
        bcat_ref[0, 384:] = bv1_ref[...]

    x = x_ref[0].astype(jnp.bfloat16)
    h_all = jnp.maximum(
        jnp.dot(x, wcat_ref[...], preferred_element_type=jnp.float32)
        + bcat_ref[...],
        0.0,
    ).astype(jnp.bfloat16)
    h = h_all[:, :384]
    v = h_all[:, 384:]
    h2 = jnp.maximum(
        lax.dot_general(h, w2t_ref[...].astype(jnp.bfloat16),
                        (((1,), (1,)), ((), ())),
                        preferred_element_type=jnp.float32)
        + b2_ref[...],
        0.0,
    ).astype(jnp.bfloat16)
    logits_ref[0] = (
        lax.dot_general(w3t_ref[...].astype(jnp.bfloat16), h2,
                        (((1,), (1,)), ((), ())),
                        preferred_element_type=jnp.float32)
        + jnp.transpose(b3_ref[...])
    )
    values_ref[0] = (
        lax.dot_general(wv2t_ref[...].astype(jnp.bfloat16), v,
                        (((1,), (1,)), ((), ())),
                        preferred_element_type=jnp.float32)
        + bv2_ref[...]
    )


def kernel(hidden_states, W1, b1, W2, b2, W3, b3, Wv1, bv1, Wv2, bv2):
    B, S, H = hidden_states.shape
    E = W3.shape[1]
    n_tiles = S // _TILE
    logits_t, values_t = pl.pallas_call(
        _router_kernel,
        grid=(B, n_tiles),
        compiler_params=pltpu.CompilerParams(
            dimension_semantics=("arbitrary", "arbitrary"),
            vmem_limit_bytes=100 * 1024 * 1024,
        ),
        in_specs=[
            pl.BlockSpec((1, _TILE, H), lambda b, i: (b, i, 0)),
            pl.BlockSpec((H, H // 2), lambda b, i: (0, 0)),
            pl.BlockSpec((H // 2,), lambda b, i: (0,)),
            pl.BlockSpec((H, H // 2), lambda b, i: (0, 0)),
            pl.BlockSpec((H // 2,), lambda b, i: (0,)),
            pl.BlockSpec((H // 4, H // 2), lambda b, i: (0, 0)),
            pl.BlockSpec((1, H // 4), lambda b, i: (0, 0)),
            pl.BlockSpec((E, H // 4), lambda b, i: (0, 0)),
            pl.BlockSpec((1, E), lambda b, i: (0, 0)),
            pl.BlockSpec((1, H // 2), lambda b, i: (0, 0)),
            pl.BlockSpec((1, 1), lambda b, i: (0, 0)),
        ],
        out_specs=[
            pl.BlockSpec((1, E, _TILE), lambda b, i: (b, 0, i)),
            pl.BlockSpec((1, 1, _TILE), lambda b, i: (b, 0, i)),
        ],
        out_shape=[
            jax.ShapeDtypeStruct((B, E, S), jnp.float32),
            jax.ShapeDtypeStruct((B, 1, S), jnp.float32),
        ],
        scratch_shapes=[
            pltpu.VMEM((H, H), jnp.bfloat16),
            pltpu.VMEM((1, H), jnp.float32),
        ],
    )(hidden_states, W1, b1, Wv1, bv1, W2.T, b2.reshape(1, -1),
      W3.T, b3.reshape(1, -1), Wv2.T, bv2.reshape(1, -1))
    logits = jnp.transpose(logits_t, (0, 2, 1))
    values = jnp.transpose(values_t, (0, 2, 1))
    return (logits, values)


# speedup vs baseline: 1.0732x; 1.0045x over previous
import jax
import jax.numpy as jnp
from jax import lax
from jax.experimental import pallas as pl
from jax.experimental.pallas import tpu as pltpu

_TILE = 2048


def _router_kernel(x_ref, w1_ref, b1_ref, wv1_ref, bv1_ref,
                   w2t_ref, b2_ref, w3t_ref, b3_ref, wv2t_ref, bv2_ref,
                   logits_ref, values_ref, wcat_ref, bcat_ref):
    @pl.when(pl.program_id(1) == 0)
    def _init():
        wcat_ref[:, :384] = w1_ref[...]
        wcat_ref[:, 384:] = wv1_ref[...]
        bcat_ref[0, :384] = b1_ref[...]
        bcat_ref[0, 384:] = bv1_ref[...]

    x = x_ref[0]
    h_all = jnp.maximum(
        jnp.dot(x, wcat_ref[...], preferred_element_type=jnp.float32)
        + bcat_ref[...],
        0.0,
    ).astype(jnp.bfloat16)
    h = h_all[:, :384]
    v = h_all[:, 384:]
    h2 = jnp.maximum(
        lax.dot_general(h, w2t_ref[...].astype(jnp.bfloat16),
                        (((1,), (1,)), ((), ())),
                        preferred_element_type=jnp.float32)
        + b2_ref[...],
        0.0,
    ).astype(jnp.bfloat16)
    logits_ref[0] = (
        lax.dot_general(w3t_ref[...].astype(jnp.bfloat16), h2,
                        (((1,), (1,)), ((), ())),
                        preferred_element_type=jnp.float32)
        + jnp.transpose(b3_ref[...])
    )
    values_ref[0] = (
        lax.dot_general(wv2t_ref[...].astype(jnp.bfloat16), v,
                        (((1,), (1,)), ((), ())),
                        preferred_element_type=jnp.float32)
        + bv2_ref[...]
    )


def kernel(hidden_states, W1, b1, W2, b2, W3, b3, Wv1, bv1, Wv2, bv2):
    B, S, H = hidden_states.shape
    E = W3.shape[1]
    n_tiles = S // _TILE
    logits_t, values_t = pl.pallas_call(
        _router_kernel,
        grid=(B, n_tiles),
        compiler_params=pltpu.CompilerParams(
            dimension_semantics=("arbitrary", "arbitrary"),
            vmem_limit_bytes=100 * 1024 * 1024,
        ),
        in_specs=[
            pl.BlockSpec((1, _TILE, H), lambda b, i: (b, i, 0)),
            pl.BlockSpec((H, H // 2), lambda b, i: (0, 0)),
            pl.BlockSpec((H // 2,), lambda b, i: (0,)),
            pl.BlockSpec((H, H // 2), lambda b, i: (0, 0)),
            pl.BlockSpec((H // 2,), lambda b, i: (0,)),
            pl.BlockSpec((H // 4, H // 2), lambda b, i: (0, 0)),
            pl.BlockSpec((1, H // 4), lambda b, i: (0, 0)),
            pl.BlockSpec((E, H // 4), lambda b, i: (0, 0)),
            pl.BlockSpec((1, E), lambda b, i: (0, 0)),
            pl.BlockSpec((1, H // 2), lambda b, i: (0, 0)),
            pl.BlockSpec((1, 1), lambda b, i: (0, 0)),
        ],
        out_specs=[
            pl.BlockSpec((1, E, _TILE), lambda b, i: (b, 0, i)),
            pl.BlockSpec((1, 1, _TILE), lambda b, i: (b, 0, i)),
        ],
        out_shape=[
            jax.ShapeDtypeStruct((B, E, S), jnp.float32),
            jax.ShapeDtypeStruct((B, 1, S), jnp.float32),
        ],
        scratch_shapes=[
            pltpu.VMEM((H, H), jnp.float32),
            pltpu.VMEM((1, H), jnp.float32),
        ],
    )(hidden_states, W1, b1, Wv1, bv1, W2.T, b2.reshape(1, -1),
      W3.T, b3.reshape(1, -1), Wv2.T, bv2.reshape(1, -1))
    logits = jnp.transpose(logits_t, (0, 2, 1))
    values = jnp.transpose(values_t, (0, 2, 1))
    return (logits, values)


# speedup vs baseline: 1.0774x; 1.0040x over previous
import jax
import jax.numpy as jnp
from jax import lax
from jax.experimental import pallas as pl
from jax.experimental.pallas import tpu as pltpu

_TILE = 2048


def _router_kernel(x_ref, w1_ref, b1_ref, wv1_ref, bv1_ref,
                   w2t_ref, b2_ref, w3t_ref, b3_ref, wv2t_ref, bv2_ref,
                   logits_ref, values_ref, wcat_ref, bcat_ref):
    @pl.when(pl.program_id(0) == 0)
    def _init():
        wcat_ref[:, :384] = w1_ref[...]
        wcat_ref[:, 384:] = wv1_ref[...]
        bcat_ref[0, :384] = b1_ref[...]
        bcat_ref[0, 384:] = bv1_ref[...]

    x = x_ref[0]
    h_all = jnp.maximum(
        jnp.dot(x, wcat_ref[...], preferred_element_type=jnp.float32)
        + bcat_ref[...],
        0.0,
    ).astype(jnp.bfloat16)
    h = h_all[:, :384]
    v = h_all[:, 384:]
    h2 = jnp.maximum(
        lax.dot_general(h, w2t_ref[...].astype(jnp.bfloat16),
                        (((1,), (1,)), ((), ())),
                        preferred_element_type=jnp.float32)
        + b2_ref[...],
        0.0,
    ).astype(jnp.bfloat16)
    logits_ref[0] = (
        lax.dot_general(w3t_ref[...].astype(jnp.bfloat16), h2,
                        (((1,), (1,)), ((), ())),
                        preferred_element_type=jnp.float32)
        + jnp.transpose(b3_ref[...])
    )
    values_ref[0] = (
        lax.dot_general(wv2t_ref[...].astype(jnp.bfloat16), v,
                        (((1,), (1,)), ((), ())),
                        preferred_element_type=jnp.float32)
        + bv2_ref[...]
    )


def kernel(hidden_states, W1, b1, W2, b2, W3, b3, Wv1, bv1, Wv2, bv2):
    B, S, H = hidden_states.shape
    E = W3.shape[1]
    n_tiles = S // _TILE
    logits_t, values_t = pl.pallas_call(
        _router_kernel,
        grid=(B * n_tiles,),
        compiler_params=pltpu.CompilerParams(
            dimension_semantics=("arbitrary",),
            vmem_limit_bytes=100 * 1024 * 1024,
        ),
        in_specs=[
            pl.BlockSpec((1, _TILE, H), lambda i: (i // n_tiles, i % n_tiles, 0)),
            pl.BlockSpec((H, H // 2), lambda i: (0, 0)),
            pl.BlockSpec((H // 2,), lambda i: (0,)),
            pl.BlockSpec((H, H // 2), lambda i: (0, 0)),
            pl.BlockSpec((H // 2,), lambda i: (0,)),
            pl.BlockSpec((H // 4, H // 2), lambda i: (0, 0)),
            pl.BlockSpec((1, H // 4), lambda i: (0, 0)),
            pl.BlockSpec((E, H // 4), lambda i: (0, 0)),
            pl.BlockSpec((1, E), lambda i: (0, 0)),
            pl.BlockSpec((1, H // 2), lambda i: (0, 0)),
            pl.BlockSpec((1, 1), lambda i: (0, 0)),
        ],
        out_specs=[
            pl.BlockSpec((1, E, _TILE), lambda i: (i // n_tiles, 0, i % n_tiles)),
            pl.BlockSpec((1, 1, _TILE), lambda i: (i // n_tiles, 0, i % n_tiles)),
        ],
        out_shape=[
            jax.ShapeDtypeStruct((B, E, S), jnp.float32),
            jax.ShapeDtypeStruct((B, 1, S), jnp.float32),
        ],
        scratch_shapes=[
            pltpu.VMEM((H, H), jnp.float32),
            pltpu.VMEM((1, H), jnp.float32),
        ],
    )(hidden_states, W1, b1, Wv1, bv1, W2.T, b2.reshape(1, -1),
      W3.T, b3.reshape(1, -1), Wv2.T, bv2.reshape(1, -1))
    logits = jnp.transpose(logits_t, (0, 2, 1))
    values = jnp.transpose(values_t, (0, 2, 1))
    return (logits, values)


# speedup vs baseline: 1.0829x; 1.0051x over previous
import jax
import jax.numpy as jnp
from jax import lax
from jax.experimental import pallas as pl
from jax.experimental.pallas import tpu as pltpu

_TILE = 2048


def _router_kernel(x_ref, w1_ref, b1_ref, wv1_ref, bv1_ref,
                   w2t_ref, b2_ref, w3t_ref, b3_ref, wv2t_ref, bv2_ref,
                   logits_ref, values_ref, wcat_ref, bcat_ref):
    @pl.when(pl.program_id(0) == 0)
    def _init():
        wcat_ref[:, :384] = w1_ref[...]
        wcat_ref[:, 384:] = wv1_ref[...]
        bcat_ref[0, :384] = b1_ref[...]
        bcat_ref[0, 384:] = bv1_ref[...]

    x = x_ref[0]
    h_all = jnp.maximum(
        jnp.dot(x, wcat_ref[...], preferred_element_type=jnp.float32)
        + bcat_ref[...],
        0.0,
    )
    h = h_all[:, :384]
    v = h_all[:, 384:]
    h2 = jnp.maximum(
        lax.dot_general(h, w2t_ref[...],
                        (((1,), (1,)), ((), ())),
                        preferred_element_type=jnp.float32)
        + b2_ref[...],
        0.0,
    )
    logits_ref[0] = (
        lax.dot_general(w3t_ref[...], h2,
                        (((1,), (1,)), ((), ())),
                        preferred_element_type=jnp.float32)
        + jnp.transpose(b3_ref[...])
    )
    values_ref[0] = (
        lax.dot_general(wv2t_ref[...], v,
                        (((1,), (1,)), ((), ())),
                        preferred_element_type=jnp.float32)
        + bv2_ref[...]
    )


def kernel(hidden_states, W1, b1, W2, b2, W3, b3, Wv1, bv1, Wv2, bv2):
    B, S, H = hidden_states.shape
    E = W3.shape[1]
    n_tiles = S // _TILE
    logits_t, values_t = pl.pallas_call(
        _router_kernel,
        grid=(B * n_tiles,),
        compiler_params=pltpu.CompilerParams(
            dimension_semantics=("arbitrary",),
            vmem_limit_bytes=100 * 1024 * 1024,
        ),
        in_specs=[
            pl.BlockSpec((1, _TILE, H), lambda i: (i // n_tiles, i % n_tiles, 0)),
            pl.BlockSpec((H, H // 2), lambda i: (0, 0)),
            pl.BlockSpec((H // 2,), lambda i: (0,)),
            pl.BlockSpec((H, H // 2), lambda i: (0, 0)),
            pl.BlockSpec((H // 2,), lambda i: (0,)),
            pl.BlockSpec((H // 4, H // 2), lambda i: (0, 0)),
            pl.BlockSpec((1, H // 4), lambda i: (0, 0)),
            pl.BlockSpec((E, H // 4), lambda i: (0, 0)),
            pl.BlockSpec((1, E), lambda i: (0, 0)),
            pl.BlockSpec((1, H // 2), lambda i: (0, 0)),
            pl.BlockSpec((1, 1), lambda i: (0, 0)),
        ],
        out_specs=[
            pl.BlockSpec((1, E, _TILE), lambda i: (i // n_tiles, 0, i % n_tiles)),
            pl.BlockSpec((1, 1, _TILE), lambda i: (i // n_tiles, 0, i % n_tiles)),
        ],
        out_shape=[
            jax.ShapeDtypeStruct((B, E, S), jnp.float32),
            jax.ShapeDtypeStruct((B, 1, S), jnp.float32),
        ],
        scratch_shapes=[
            pltpu.VMEM((H, H), jnp.float32),
            pltpu.VMEM((1, H), jnp.float32),
        ],
    )(hidden_states, W1, b1, Wv1, bv1, W2.T, b2.reshape(1, -1),
      W3.T, b3.reshape(1, -1), Wv2.T, bv2.reshape(1, -1))
    logits = jnp.transpose(logits_t, (0, 2, 1))
    values = jnp.transpose(values_t, (0, 2, 1))
    return (logits, values)
